# speedup vs baseline: 2.7292x; 2.7292x over previous
import jax
import jax.numpy as jnp
from jax.experimental import pallas as pl
from jax.experimental.pallas import tpu as pltpu

B = 1024
IN = 64
H = 512
MAX_PONDER = 20
EPS = 0.01


def _act_body(x_ref, wxt_ref, wflag_ref, bias_ref, whht_ref, wp_ref, bp_ref,
              wfc_ref, bfc_ref, out_ref, pc_ref,
              base_ref, hx_ref, accum_hx_ref, accum_h_ref, spc_ref, sc_ref,
              done_ref):
    base_ref[:] = (
        jnp.dot(x_ref[:], wxt_ref[:], preferred_element_type=jnp.float32)
        + bias_ref[:]
    )
    hx_ref[:] = jnp.zeros((B, H), jnp.float32)
    accum_hx_ref[:] = jnp.zeros((B, H), jnp.float32)
    accum_h_ref[:] = jnp.zeros((B, 1), jnp.float32)
    spc_ref[:] = jnp.zeros((B, 1), jnp.float32)
    sc_ref[:] = jnp.zeros((B, 1), jnp.float32)
    done_ref[0] = 0

    def step(i, carry):
        @pl.when(done_ref[0] == 0)
        def _():
            accum_h = accum_h_ref[:]
            sel = accum_h < (1.0 - EPS)
            spc_ref[:] = jnp.where(sel, accum_h, spc_ref[:])
            flag = i.astype(jnp.float32)
            h_new = jnp.tanh(
                base_ref[:]
                + flag * wflag_ref[:]
                + jnp.dot(hx_ref[:], whht_ref[:],
                          preferred_element_type=jnp.float32)
            )
            hx = jnp.where(sel, h_new, hx_ref[:])
            hx_ref[:] = hx
            h = jax.nn.sigmoid(
                jnp.sum(hx * wp_ref[:], axis=1, keepdims=True) + bp_ref[:]
            )
            accum_h_new = accum_h + jnp.where(sel, h, 0.0)
            p = h - jnp.maximum(accum_h_new - 1.0, 0.0)
            accum_hx_ref[:] = accum_hx_ref[:] + jnp.where(
                sel, (1.0 + p) * hx, 0.0)
            accum_h_ref[:] = accum_h_new
            sc_ref[:] = sc_ref[:] + jnp.where(sel, 1.0, 0.0)
            all_halted = jnp.min(accum_h_new) >= (1.0 - EPS)
            done_ref[0] = all_halted.astype(jnp.int32)
        return carry

    jax.lax.fori_loop(0, MAX_PONDER, step, 0)

    hx_final = accum_hx_ref[:] / sc_ref[:]
    out_ref[:] = (
        jnp.sum(hx_final * wfc_ref[:], axis=1, keepdims=True) + bfc_ref[:]
    )
    pc_ref[:] = -spc_ref[:]


@jax.jit
def _act_kernel(x, wxt, wflag, bias, whht, wp, bp, wfc, bfc):
    out, pc = pl.pallas_call(
        _act_body,
        out_shape=(
            jax.ShapeDtypeStruct((B, 1), jnp.float32),
            jax.ShapeDtypeStruct((B, 1), jnp.float32),
        ),
        scratch_shapes=[
            pltpu.VMEM((B, H), jnp.float32),
            pltpu.VMEM((B, H), jnp.float32),
            pltpu.VMEM((B, H), jnp.float32),
            pltpu.VMEM((B, 1), jnp.float32),
            pltpu.VMEM((B, 1), jnp.float32),
            pltpu.VMEM((B, 1), jnp.float32),
            pltpu.SMEM((1,), jnp.int32),
        ],
    )(x, wxt, wflag, bias, whht, wp, bp, wfc, bfc)
    return out[:, 0], pc[:, 0]


def kernel(x, W_ih, b_ih, W_hh, b_hh, W_p, b_p, W_fc, b_fc):
    wxt = W_ih[:, :IN].T
    wflag = W_ih[:, IN][None, :]
    bias = (b_ih + b_hh)[None, :]
    whht = W_hh.T
    wp = W_p
    bp = b_p[None, :]
    wfc = W_fc
    bfc = b_fc[None, :]
    return _act_kernel(x, wxt, wflag, bias, whht, wp, bp, wfc, bfc)


# speedup vs baseline: 2.7586x; 1.0108x over previous
import jax
import jax.numpy as jnp
from jax.experimental import pallas as pl
from jax.experimental.pallas import tpu as pltpu

B = 1024
IN = 64
H = 512
MAX_PONDER = 20
EPS = 0.01


def _act_body(x_ref, wxt_ref, wflag_ref, bias_ref, whht_ref, wp_ref, bp_ref,
              wfc_ref, bfc_ref, out_ref, pc_ref,
              base_ref, hx_ref, accum_hx_ref, accum_h_ref, spc_ref, sc_ref):
    base_ref[:] = (
        jnp.dot(x_ref[:], wxt_ref[:], preferred_element_type=jnp.float32)
        + bias_ref[:]
    )
    hx_ref[:] = jnp.zeros((B, H), jnp.float32)
    accum_hx_ref[:] = jnp.zeros((B, H), jnp.float32)
    accum_h_ref[:] = jnp.zeros((B, 1), jnp.float32)
    spc_ref[:] = jnp.zeros((B, 1), jnp.float32)
    sc_ref[:] = jnp.zeros((B, 1), jnp.float32)

    def cond(carry):
        i, done = carry
        return jnp.logical_and(i < MAX_PONDER, done == 0)

    def step(carry):
        i, _ = carry
        accum_h = accum_h_ref[:]
        sel = accum_h < (1.0 - EPS)
        spc_ref[:] = jnp.where(sel, accum_h, spc_ref[:])
        flag = i.astype(jnp.float32)
        h_new = jnp.tanh(
            base_ref[:]
            + flag * wflag_ref[:]
            + jnp.dot(hx_ref[:], whht_ref[:],
                      preferred_element_type=jnp.float32)
        )
        hx = jnp.where(sel, h_new, hx_ref[:])
        hx_ref[:] = hx
        h = jax.nn.sigmoid(
            jnp.sum(hx * wp_ref[:], axis=1, keepdims=True) + bp_ref[:]
        )
        accum_h_new = accum_h + jnp.where(sel, h, 0.0)
        p = h - jnp.maximum(accum_h_new - 1.0, 0.0)
        accum_hx_ref[:] = accum_hx_ref[:] + jnp.where(
            sel, (1.0 + p) * hx, 0.0)
        accum_h_ref[:] = accum_h_new
        sc_ref[:] = sc_ref[:] + jnp.where(sel, 1.0, 0.0)
        all_halted = jnp.min(accum_h_new) >= (1.0 - EPS)
        return i + 1, all_halted.astype(jnp.int32)

    jax.lax.while_loop(cond, step, (0, 0))

    hx_final = accum_hx_ref[:] / sc_ref[:]
    out_ref[:] = (
        jnp.sum(hx_final * wfc_ref[:], axis=1, keepdims=True) + bfc_ref[:]
    )
    pc_ref[:] = -spc_ref[:]


@jax.jit
def _act_kernel(x, wxt, wflag, bias, whht, wp, bp, wfc, bfc):
    out, pc = pl.pallas_call(
        _act_body,
        out_shape=(
            jax.ShapeDtypeStruct((B, 1), jnp.float32),
            jax.ShapeDtypeStruct((B, 1), jnp.float32),
        ),
        scratch_shapes=[
            pltpu.VMEM((B, H), jnp.float32),
            pltpu.VMEM((B, H), jnp.float32),
            pltpu.VMEM((B, H), jnp.float32),
            pltpu.VMEM((B, 1), jnp.float32),
            pltpu.VMEM((B, 1), jnp.float32),
            pltpu.VMEM((B, 1), jnp.float32),
        ],
    )(x, wxt, wflag, bias, whht, wp, bp, wfc, bfc)
    return out[:, 0], pc[:, 0]


def kernel(x, W_ih, b_ih, W_hh, b_hh, W_p, b_p, W_fc, b_fc):
    wxt = W_ih[:, :IN].T
    wflag = W_ih[:, IN][None, :]
    bias = (b_ih + b_hh)[None, :]
    whht = W_hh.T
    wp = W_p
    bp = b_p[None, :]
    wfc = W_fc
    bfc = b_fc[None, :]
    return _act_kernel(x, wxt, wflag, bias, whht, wp, bp, wfc, bfc)


# speedup vs baseline: 3.5662x; 1.2927x over previous
import jax
import jax.numpy as jnp
from jax.experimental import pallas as pl
from jax.experimental.pallas import tpu as pltpu

B = 1024
IN = 64
H = 512
MAX_PONDER = 20
EPS = 0.01

_DN_T = (((1,), (1,)), ((), ()))


def _act_body(x_ref, wih_ref, bias_ref, whh_ref, wp_ref, bp_ref,
              wfc_ref, bfc_ref, out_ref, pc_ref,
              base_ref, hx_ref, accum_hx_ref, accum_h_ref, spc_ref, sc_ref):
    f32 = jnp.float32
    base_ref[:] = jax.lax.dot_general(
        x_ref[:], wih_ref[:, :IN], _DN_T, preferred_element_type=f32
    ) + bias_ref[:]
    onehot = (jax.lax.broadcasted_iota(jnp.int32, (1, IN + 1), 1) == IN)
    wflag = jax.lax.dot_general(
        onehot.astype(f32), wih_ref[:], _DN_T, preferred_element_type=f32)

    hx0 = jnp.tanh(base_ref[:])
    hx_ref[:] = hx0
    h0 = jax.nn.sigmoid(
        jnp.sum(hx0 * wp_ref[:], axis=1, keepdims=True) + bp_ref[:])
    p0 = h0 - jnp.maximum(h0 - 1.0, 0.0)
    accum_hx_ref[:] = (1.0 + p0) * hx0
    accum_h_ref[:] = h0
    spc_ref[:] = jnp.zeros((B, 1), f32)
    sc_ref[:] = jnp.ones((B, 1), f32)
    done0 = (jnp.min(h0) >= (1.0 - EPS)).astype(jnp.int32)

    def cond(carry):
        i, done = carry
        return jnp.logical_and(i < MAX_PONDER, done == 0)

    def step(carry):
        i, _ = carry
        accum_h = accum_h_ref[:]
        sel = accum_h < (1.0 - EPS)
        spc_ref[:] = jnp.where(sel, accum_h, spc_ref[:])
        flag = i.astype(f32)
        h_new = jnp.tanh(
            base_ref[:]
            + flag * wflag
            + jax.lax.dot_general(hx_ref[:], whh_ref[:], _DN_T,
                                  preferred_element_type=f32)
        )
        hx = jnp.where(sel, h_new, hx_ref[:])
        hx_ref[:] = hx
        h = jax.nn.sigmoid(
            jnp.sum(hx * wp_ref[:], axis=1, keepdims=True) + bp_ref[:]
        )
        accum_h_new = accum_h + jnp.where(sel, h, 0.0)
        p = h - jnp.maximum(accum_h_new - 1.0, 0.0)
        accum_hx_ref[:] = accum_hx_ref[:] + jnp.where(
            sel, (1.0 + p) * hx, 0.0)
        accum_h_ref[:] = accum_h_new
        sc_ref[:] = sc_ref[:] + jnp.where(sel, 1.0, 0.0)
        all_halted = jnp.min(accum_h_new) >= (1.0 - EPS)
        return i + 1, all_halted.astype(jnp.int32)

    jax.lax.while_loop(cond, step, (1, done0))

    hx_final = accum_hx_ref[:] / sc_ref[:]
    out_ref[:] = (
        jnp.sum(hx_final * wfc_ref[:], axis=1, keepdims=True) + bfc_ref[:]
    )
    pc_ref[:] = -spc_ref[:]


@jax.jit
def _act_kernel(x, W_ih, b_ih, W_hh, b_hh, W_p, b_p, W_fc, b_fc):
    bias = (b_ih + b_hh)[None, :]
    bp = b_p[None, :]
    bfc = b_fc[None, :]
    out, pc = pl.pallas_call(
        _act_body,
        out_shape=(
            jax.ShapeDtypeStruct((B, 1), jnp.float32),
            jax.ShapeDtypeStruct((B, 1), jnp.float32),
        ),
        scratch_shapes=[
            pltpu.VMEM((B, H), jnp.float32),
            pltpu.VMEM((B, H), jnp.float32),
            pltpu.VMEM((B, H), jnp.float32),
            pltpu.VMEM((B, 1), jnp.float32),
            pltpu.VMEM((B, 1), jnp.float32),
            pltpu.VMEM((B, 1), jnp.float32),
        ],
    )(x, W_ih, bias, W_hh, W_p, bp, W_fc, bfc)
    return out[:, 0], pc[:, 0]


def kernel(x, W_ih, b_ih, W_hh, b_hh, W_p, b_p, W_fc, b_fc):
    return _act_kernel(x, W_ih, b_ih, W_hh, b_hh, W_p, b_p, W_fc, b_fc)


# speedup vs baseline: 4.9770x; 1.3956x over previous
import jax
import jax.numpy as jnp
from jax.experimental import pallas as pl
from jax.experimental.pallas import tpu as pltpu

B = 1024
IN = 64
H = 512
MAX_PONDER = 20
EPS = 0.01

_DN_T = (((1,), (1,)), ((), ()))


def _act_body(x_ref, wih_ref, bih_ref, whh_ref, bhh_ref, wp_ref, bp_ref,
              wfc_ref, bfc_ref, out_ref, pc_ref,
              base_ref, hx_ref, accum_hx_ref, accum_h_ref, spc_ref, sc_ref):
    f32 = jnp.float32
    bias = jnp.reshape(bih_ref[:], (1, H)) + jnp.reshape(bhh_ref[:], (1, H))
    bp = bp_ref[0]
    base_ref[:] = jax.lax.dot_general(
        x_ref[:], wih_ref[:, :IN], _DN_T, preferred_element_type=f32
    ) + bias
    onehot = (jax.lax.broadcasted_iota(jnp.int32, (1, IN + 1), 1) == IN)
    wflag = jax.lax.dot_general(
        onehot.astype(f32), wih_ref[:], _DN_T, preferred_element_type=f32)

    hx0 = jnp.tanh(base_ref[:])
    hx_ref[:] = hx0
    h0 = jax.nn.sigmoid(
        jnp.sum(hx0 * wp_ref[:], axis=1, keepdims=True) + bp)
    p0 = h0 - jnp.maximum(h0 - 1.0, 0.0)
    accum_hx_ref[:] = (1.0 + p0) * hx0
    accum_h_ref[:] = h0
    spc_ref[:] = jnp.zeros((B, 1), f32)
    sc_ref[:] = jnp.ones((B, 1), f32)
    done0 = (jnp.min(h0) >= (1.0 - EPS)).astype(jnp.int32)

    def cond(carry):
        i, done = carry
        return jnp.logical_and(i < MAX_PONDER, done == 0)

    def step(carry):
        i, _ = carry
        accum_h = accum_h_ref[:]
        sel = accum_h < (1.0 - EPS)
        spc_ref[:] = jnp.where(sel, accum_h, spc_ref[:])
        flag = i.astype(f32)
        hx = jnp.tanh(
            base_ref[:]
            + flag * wflag
            + jax.lax.dot_general(hx_ref[:], whh_ref[:], _DN_T,
                                  preferred_element_type=f32)
        )
        hx_ref[:] = hx
        h = jax.nn.sigmoid(
            jnp.sum(hx * wp_ref[:], axis=1, keepdims=True) + bp
        )
        accum_h_new = accum_h + jnp.where(sel, h, 0.0)
        p = h - jnp.maximum(accum_h_new - 1.0, 0.0)
        accum_hx_ref[:] = accum_hx_ref[:] + jnp.where(
            sel, (1.0 + p) * hx, 0.0)
        accum_h_ref[:] = accum_h_new
        sc_ref[:] = sc_ref[:] + jnp.where(sel, 1.0, 0.0)
        all_halted = jnp.min(accum_h_new) >= (1.0 - EPS)
        return i + 1, all_halted.astype(jnp.int32)

    jax.lax.while_loop(cond, step, (1, done0))

    hx_final = accum_hx_ref[:] / sc_ref[:]
    out_col = (
        jnp.sum(hx_final * wfc_ref[:], axis=1, keepdims=True) + bfc_ref[0]
    )
    out_ref[:] = jnp.reshape(out_col, (B,))
    pc_ref[:] = jnp.reshape(-spc_ref[:], (B,))


@jax.jit
def _act_kernel(x, W_ih, b_ih, W_hh, b_hh, W_p, b_p, W_fc, b_fc):
    return pl.pallas_call(
        _act_body,
        out_shape=(
            jax.ShapeDtypeStruct((B,), jnp.float32),
            jax.ShapeDtypeStruct((B,), jnp.float32),
        ),
        in_specs=[
            pl.BlockSpec(memory_space=pltpu.VMEM),
            pl.BlockSpec(memory_space=pltpu.VMEM),
            pl.BlockSpec(memory_space=pltpu.VMEM),
            pl.BlockSpec(memory_space=pltpu.VMEM),
            pl.BlockSpec(memory_space=pltpu.VMEM),
            pl.BlockSpec(memory_space=pltpu.VMEM),
            pl.BlockSpec(memory_space=pltpu.SMEM),
            pl.BlockSpec(memory_space=pltpu.VMEM),
            pl.BlockSpec(memory_space=pltpu.SMEM),
        ],
        scratch_shapes=[
            pltpu.VMEM((B, H), jnp.float32),
            pltpu.VMEM((B, H), jnp.float32),
            pltpu.VMEM((B, H), jnp.float32),
            pltpu.VMEM((B, 1), jnp.float32),
            pltpu.VMEM((B, 1), jnp.float32),
            pltpu.VMEM((B, 1), jnp.float32),
        ],
    )(x, W_ih, b_ih, W_hh, b_hh, W_p, b_p, W_fc, b_fc)


def kernel(x, W_ih, b_ih, W_hh, b_hh, W_p, b_p, W_fc, b_fc):
    return _act_kernel(x, W_ih, b_ih, W_hh, b_hh, W_p, b_p, W_fc, b_fc)
